# Initial kernel scaffold; baseline (speedup 1.0000x reference)
import jax
import jax.numpy as jnp
from jax.experimental import pallas as pl


def kernel(proc_times, assigned, machine_times, job_embed, fc_w, fc_b):
    raise NotImplementedError("write your pallas kernel here")


# speedup vs baseline: 49.8212x; 49.8212x over previous
import functools

import jax
import jax.numpy as jnp
from jax import lax
from jax.experimental import pallas as pl
from jax.experimental.pallas import tpu as pltpu
from jax.experimental.pallas import tpu_sc as plsc

_LANES = 16
_NUM_TILES = 32


def _sc_body(n_jobs, rows_per_tile, chunk_rows, vocab, emb_dim,
             pt_hbm, asg_hbm, emb_hbm, w_hbm, b_hbm, out_hbm,
             emb_v, w_v, b_v, accbuf, etab, pt_buf, asg_buf, e_buf):
    tile = lax.axis_index("s") * 2 + lax.axis_index("c")
    iota = lax.iota(jnp.int32, _LANES)

    pltpu.sync_copy(emb_hbm, emb_v)
    pltpu.sync_copy(w_hbm, w_v)
    pltpu.sync_copy(b_hbm, b_v)
    for v in range(vocab):
        acc = jnp.zeros((_LANES,), jnp.float32)
        for k in range(emb_dim // _LANES):
            acc = acc + (emb_v[pl.ds(v * emb_dim + k * _LANES, _LANES)]
                         * w_v[pl.ds(k * _LANES, _LANES)])
        accbuf[pl.ds(v * _LANES, _LANES)] = acc
    tvec = jnp.zeros((_LANES,), jnp.float32)
    for l in range(_LANES):
        tvec = tvec + plsc.load_gather(accbuf, [iota * _LANES + l])
    tvec = jnp.where(iota < vocab, jnp.exp(tvec + b_v[...]), 0.0)
    etab[...] = tvec

    chunk_elems = chunk_rows * n_jobs
    n_chunks = rows_per_tile // chunk_rows
    n_groups = chunk_rows // _LANES
    base = tile * (rows_per_tile * n_jobs)

    for chunk in range(n_chunks):
        off = base + chunk * chunk_elems
        pltpu.sync_copy(pt_hbm.at[pl.ds(off, chunk_elems)], pt_buf)
        pltpu.sync_copy(asg_hbm.at[pl.ds(off, chunk_elems)], asg_buf)

        for g in range(n_groups):
            idx0 = (g * _LANES + iota) * n_jobs

            def pass1(i, acc, idx0=idx0):
                for u in range(4):
                    idx = idx0 + (i * 4 + u)
                    ptv = plsc.load_gather(pt_buf, [idx])
                    av = plsc.load_gather(asg_buf, [idx])
                    ev = plsc.load_gather(etab, [ptv])
                    ev = jnp.where(av > 0, 0.0, ev)
                    acc = acc + ev
                    plsc.store_scatter(e_buf, [idx], ev)
                return acc

            ssum = lax.fori_loop(0, n_jobs // 4, pass1,
                                 jnp.zeros((_LANES,), jnp.float32))
            recip = 1.0 / ssum

            def pass2(i, carry, idx0=idx0, recip=recip):
                for u in range(4):
                    idx = idx0 + (i * 4 + u)
                    ev = plsc.load_gather(e_buf, [idx])
                    plsc.store_scatter(e_buf, [idx], ev * recip)
                return carry

            lax.fori_loop(0, n_jobs // 4, pass2, 0)

        pltpu.sync_copy(e_buf, out_hbm.at[pl.ds(off, chunk_elems)])


@functools.partial(jax.jit, static_argnames=("b", "n_jobs", "vocab",
                                             "emb_dim"))
def _sc_call(pt, asg, emb, w, b16, *, b, n_jobs, vocab, emb_dim):
    rows_per_tile = b // _NUM_TILES
    chunk_rows = 64
    chunk_elems = chunk_rows * n_jobs
    mesh = plsc.VectorSubcoreMesh(core_axis_name="c", subcore_axis_name="s")
    body = functools.partial(_sc_body, n_jobs, rows_per_tile, chunk_rows,
                             vocab, emb_dim)
    return pl.kernel(
        body,
        out_type=jax.ShapeDtypeStruct((b * n_jobs,), jnp.float32),
        mesh=mesh,
        compiler_params=pltpu.CompilerParams(needs_layout_passes=False),
        scratch_types=[
            pltpu.VMEM((vocab * emb_dim,), jnp.float32),
            pltpu.VMEM((emb_dim,), jnp.float32),
            pltpu.VMEM((_LANES,), jnp.float32),
            pltpu.VMEM((_LANES * _LANES,), jnp.float32),
            pltpu.VMEM((_LANES,), jnp.float32),
            pltpu.VMEM((chunk_elems,), jnp.int32),
            pltpu.VMEM((chunk_elems,), jnp.int32),
            pltpu.VMEM((chunk_elems,), jnp.float32),
        ],
    )(pt, asg, emb, w, b16)


def kernel(proc_times, assigned, machine_times, job_embed, fc_w, fc_b):
    b, n_jobs = proc_times.shape
    vocab, emb_dim = job_embed.shape
    out = _sc_call(
        proc_times.reshape(-1),
        assigned.reshape(-1),
        job_embed.reshape(-1),
        fc_w.reshape(-1),
        jnp.broadcast_to(fc_b, (_LANES,)),
        b=b, n_jobs=n_jobs, vocab=vocab, emb_dim=emb_dim,
    )
    return out.reshape(b, n_jobs)


# speedup vs baseline: 73.6505x; 1.4783x over previous
import functools

import jax
import jax.numpy as jnp
from jax import lax
from jax.experimental import pallas as pl
from jax.experimental.pallas import tpu as pltpu
from jax.experimental.pallas import tpu_sc as plsc

_LANES = 16
_NUM_TILES = 32


def _sc_body(n_jobs, rows_per_tile, chunk_rows, vocab, emb_dim,
             pt_hbm, asg_hbm, emb_hbm, w_hbm, b_hbm, out_hbm,
             emb_v, w_v, b_v, accbuf, etab, pt_buf, asg_buf, e_buf, out_buf):
    tile = lax.axis_index("s") * 2 + lax.axis_index("c")
    iota = lax.iota(jnp.int32, _LANES)

    pltpu.sync_copy(emb_hbm, emb_v)
    pltpu.sync_copy(w_hbm, w_v)
    pltpu.sync_copy(b_hbm, b_v)
    for v in range(vocab):
        acc = jnp.zeros((_LANES,), jnp.float32)
        for k in range(emb_dim // _LANES):
            acc = acc + (emb_v[pl.ds(v * emb_dim + k * _LANES, _LANES)]
                         * w_v[pl.ds(k * _LANES, _LANES)])
        accbuf[pl.ds(v * _LANES, _LANES)] = acc
    tvec = jnp.zeros((_LANES,), jnp.float32)
    for l in range(_LANES):
        tvec = tvec + plsc.load_gather(accbuf, [iota * _LANES + l])
    tvec = jnp.where(iota < vocab, jnp.exp(tvec + b_v[...]), 0.0)
    etab[...] = tvec

    chunk_elems = chunk_rows * n_jobs
    n_chunks = rows_per_tile // chunk_rows
    n_groups = chunk_rows // _LANES
    base = tile * (rows_per_tile * n_jobs)

    for chunk in range(n_chunks):
        off = base + chunk * chunk_elems
        pltpu.sync_copy(pt_hbm.at[pl.ds(off, chunk_elems)], pt_buf)
        pltpu.sync_copy(asg_hbm.at[pl.ds(off, chunk_elems)], asg_buf)

        for g in range(n_groups):
            idx0 = (g * _LANES + iota) * n_jobs

            @plsc.parallel_loop(0, n_jobs, unroll=8,
                                carry=jnp.zeros((_LANES,), jnp.float32))
            def pass1(j, acc, idx0=idx0):
                idx = idx0 + j
                ptv = plsc.load_gather(pt_buf, [idx])
                av = plsc.load_gather(asg_buf, [idx])
                ev = plsc.load_gather(etab, [ptv])
                ev = jnp.where(av > 0, 0.0, ev)
                plsc.store_scatter(e_buf, [idx], ev)
                return acc + ev

            recip = 1.0 / pass1

            @plsc.parallel_loop(0, n_jobs, unroll=8)
            def pass2(j, idx0=idx0, recip=recip):
                idx = idx0 + j
                ev = plsc.load_gather(e_buf, [idx])
                plsc.store_scatter(out_buf, [idx], ev * recip)

        pltpu.sync_copy(out_buf, out_hbm.at[pl.ds(off, chunk_elems)])


@functools.partial(jax.jit, static_argnames=("b", "n_jobs", "vocab",
                                             "emb_dim"))
def _sc_call(pt, asg, emb, w, b16, *, b, n_jobs, vocab, emb_dim):
    rows_per_tile = b // _NUM_TILES
    chunk_rows = 64
    chunk_elems = chunk_rows * n_jobs
    mesh = plsc.VectorSubcoreMesh(core_axis_name="c", subcore_axis_name="s")
    body = functools.partial(_sc_body, n_jobs, rows_per_tile, chunk_rows,
                             vocab, emb_dim)
    return pl.kernel(
        body,
        out_type=jax.ShapeDtypeStruct((b * n_jobs,), jnp.float32),
        mesh=mesh,
        compiler_params=pltpu.CompilerParams(needs_layout_passes=False),
        scratch_types=[
            pltpu.VMEM((vocab * emb_dim,), jnp.float32),
            pltpu.VMEM((emb_dim,), jnp.float32),
            pltpu.VMEM((_LANES,), jnp.float32),
            pltpu.VMEM((_LANES * _LANES,), jnp.float32),
            pltpu.VMEM((_LANES,), jnp.float32),
            pltpu.VMEM((chunk_elems,), jnp.int32),
            pltpu.VMEM((chunk_elems,), jnp.int32),
            pltpu.VMEM((chunk_elems,), jnp.float32),
            pltpu.VMEM((chunk_elems,), jnp.float32),
        ],
    )(pt, asg, emb, w, b16)


def kernel(proc_times, assigned, machine_times, job_embed, fc_w, fc_b):
    b, n_jobs = proc_times.shape
    vocab, emb_dim = job_embed.shape
    out = _sc_call(
        proc_times.reshape(-1),
        assigned.reshape(-1),
        job_embed.reshape(-1),
        fc_w.reshape(-1),
        jnp.broadcast_to(fc_b, (_LANES,)),
        b=b, n_jobs=n_jobs, vocab=vocab, emb_dim=emb_dim,
    )
    return out.reshape(b, n_jobs)
